# baseline (speedup 1.0000x reference)
import jax
import jax.numpy as jnp
from jax import lax
from jax.experimental import pallas as pl
from jax.experimental.pallas import tpu as pltpu

_B, _N, _C = 32, 8732, 21
_NP = 8832
_LR, _LC = 148, 236
_CR = _LC // 4
_INF_BITS = 0x7F800000


def _fused(conf_ref, ct_ref, aux_ref, out_ref, lm_s, sc_s):
    i = pl.program_id(0)

    @pl.when(i == 0)
    def _zero():
        lm_s[...] = jnp.zeros((_B, _NP), jnp.float32)
        sc_s[...] = jnp.zeros((8, 128), jnp.float32)

    @pl.when(i < _B)
    def _stage1():
        ct_row = ct_ref[pl.ds(i, 1), :]
        posf = (ct_row > 0).astype(jnp.float32)

        Xt = jnp.transpose(conf_ref[0])
        sub = lax.broadcasted_iota(jnp.int32, (_C, _N), 0)
        ones_row = jnp.ones((1, _C), jnp.float32)
        hi = lax.Precision.DEFAULT
        Em = jnp.exp(Xt)
        Z = jnp.dot(ones_row, Em, precision=hi)
        pick = jnp.dot(ones_row,
                       jnp.where(sub == ct_row, Xt, 0.0),
                       precision=hi)
        ce = jnp.log(Z) - pick
        lm_s[pl.ds(i, 1), 0:_N] = jnp.where(posf > 0, 0.0, ce)
        spce_b = jnp.sum(ce * posf)

        ld = aux_ref[0, 0] - aux_ref[0, 1]
        ad = jnp.abs(ld)
        y = jnp.where(ad < 1.0, 0.5 * ld * ld, ad - 0.5)
        r = (lax.broadcasted_iota(jnp.int32, (_LC, _CR), 0) // 4
             == lax.broadcasted_iota(jnp.int32, (_LC, _CR), 1)
             ).astype(jnp.float32)
        s59 = jnp.dot(y, r, precision=hi)
        posf59 = (aux_ref[0, 2, :, 0:_CR] > 0).astype(jnp.float32)
        loc_b = jnp.sum(s59 * posf59)

        row = lax.broadcasted_iota(jnp.int32, (8, 128), 0)
        sc_s[...] += (jnp.where(row == 0, loc_b, 0.0)
                      + jnp.where(row == 1, spce_b, 0.0))

    @pl.when(i == _B)
    def _stage2():
        lm = lm_s[...]
        posm = (ct_ref[...] > 0).astype(jnp.float32)
        npb = jnp.sum(posm, axis=1, keepdims=True)
        kf = jnp.clip(3.0 * npb, 1.0, float(_N - 1))
        bits = lax.bitcast_convert_type(lm, jnp.int32)

        def step(_, lohi):
            lo, hi2 = lohi
            mid = lo + ((hi2 - lo + 1) >> 1)
            cnt = jnp.sum((bits >= mid).astype(jnp.float32), axis=1,
                          keepdims=True)
            ge = cnt >= kf
            return jnp.where(ge, mid, lo), jnp.where(ge, hi2, mid - 1)

        lo0 = jnp.zeros((_B, 1), jnp.int32)
        hi0 = jnp.full((_B, 1), _INF_BITS, jnp.int32)
        lo, _ = lax.fori_loop(0, 31, step, (lo0, hi0))
        tau = lax.bitcast_convert_type(lo, jnp.float32)

        gt = lm > tau
        cnt_gt = jnp.sum(gt.astype(jnp.float32), axis=1, keepdims=True)
        sum_gt = jnp.sum(jnp.where(gt, lm, 0.0), axis=1, keepdims=True)
        topk = sum_gt + (kf - cnt_gt) * tau
        num_matched = jnp.sum(npb)
        conf_loss = sc_s[1, 0] + jnp.sum(topk)
        out_ref[0, 0] = (sc_s[0, 0] + conf_loss) / num_matched


def kernel(loc_preds, loc_targets, conf_preds, conf_targets):
    ct59f = jnp.pad(conf_targets.reshape(_B, _LR, _CR).astype(jnp.float32),
                    ((0, 0), (0, 0), (0, _LC - _CR)))
    aux = jnp.concatenate(
        [loc_preds.reshape(_B, 1, _LR, _LC),
         loc_targets.reshape(_B, 1, _LR, _LC),
         ct59f.reshape(_B, 1, _LR, _LC)], axis=1)

    def ix3(i):
        return (jnp.minimum(i, _B - 1), 0, 0)

    def ix4(i):
        return (jnp.minimum(i, _B - 1), 0, 0, 0)

    loss = pl.pallas_call(
        _fused,
        grid=(_B + 1,),
        in_specs=[
            pl.BlockSpec((1, _N, _C), ix3),
            pl.BlockSpec((_B, _N), lambda i: (0, 0)),
            pl.BlockSpec((1, 3, _LR, _LC), ix4),
        ],
        out_specs=pl.BlockSpec(memory_space=pltpu.SMEM),
        out_shape=jax.ShapeDtypeStruct((1, 1), jnp.float32),
        scratch_shapes=[
            pltpu.VMEM((_B, _NP), jnp.float32),
            pltpu.VMEM((8, 128), jnp.float32),
        ],
    )(conf_preds, conf_targets, aux)
    return loss[0, 0]


# speedup vs baseline: 1.0317x; 1.0317x over previous
import jax
import jax.numpy as jnp
from jax import lax
from jax.experimental import pallas as pl
from jax.experimental.pallas import tpu as pltpu

_B, _N, _C = 32, 8732, 21
_NP = 8832
_LR, _LC = 148, 236
_CR = _LC // 4
_INF_BITS = 0x7F800000


def _fused(conf_ref, ct_ref, lpt_ref, ct59_ref, out_ref, lm_s, sc_s):
    i = pl.program_id(0)

    @pl.when(i == 0)
    def _zero():
        lm_s[...] = jnp.zeros((_B, _NP), jnp.float32)
        sc_s[...] = jnp.zeros((8, 128), jnp.float32)

    @pl.when(i < _B)
    def _stage1():
        ct_row = ct_ref[pl.ds(i, 1), :]
        posf = (ct_row > 0).astype(jnp.float32)

        Xt = jnp.transpose(conf_ref[0])
        sub = lax.broadcasted_iota(jnp.int32, (_C, _N), 0)
        ones_row = jnp.ones((1, _C), jnp.float32)
        hi = lax.Precision.DEFAULT
        Em = jnp.exp(Xt)
        Z = jnp.dot(ones_row, Em, precision=hi)
        pick = jnp.dot(ones_row,
                       jnp.where(sub == ct_row, Xt, 0.0),
                       precision=hi)
        ce = jnp.log(Z) - pick
        lm_s[pl.ds(i, 1), 0:_N] = jnp.where(posf > 0, 0.0, ce)
        spce_b = jnp.sum(ce * posf)

        ld = lpt_ref[0, 0] - lpt_ref[0, 1]
        ad = jnp.abs(ld)
        y = jnp.where(ad < 1.0, 0.5 * ld * ld, ad - 0.5)
        r = (lax.broadcasted_iota(jnp.int32, (_LC, _CR), 0) // 4
             == lax.broadcasted_iota(jnp.int32, (_LC, _CR), 1)
             ).astype(jnp.float32)
        s59 = jnp.dot(y, r, precision=hi)
        posf59 = (ct59_ref[0] > 0).astype(jnp.float32)
        loc_b = jnp.sum(s59 * posf59)

        row = lax.broadcasted_iota(jnp.int32, (8, 128), 0)
        sc_s[...] += (jnp.where(row == 0, loc_b, 0.0)
                      + jnp.where(row == 1, spce_b, 0.0))

    @pl.when(i == _B)
    def _stage2():
        lm = lm_s[...]
        posm = (ct_ref[...] > 0).astype(jnp.float32)
        npb = jnp.sum(posm, axis=1, keepdims=True)
        kf = jnp.clip(3.0 * npb, 1.0, float(_N - 1))
        bits = lax.bitcast_convert_type(lm, jnp.int32)

        def step(_, lohi):
            lo, hi2 = lohi
            mid = lo + ((hi2 - lo + 1) >> 1)
            cnt = jnp.sum((bits >= mid).astype(jnp.float32), axis=1,
                          keepdims=True)
            ge = cnt >= kf
            return jnp.where(ge, mid, lo), jnp.where(ge, hi2, mid - 1)

        lo0 = jnp.zeros((_B, 1), jnp.int32)
        hi0 = jnp.full((_B, 1), _INF_BITS, jnp.int32)
        lo, _ = lax.fori_loop(0, 31, step, (lo0, hi0))
        tau = lax.bitcast_convert_type(lo, jnp.float32)

        gt = lm > tau
        cnt_gt = jnp.sum(gt.astype(jnp.float32), axis=1, keepdims=True)
        sum_gt = jnp.sum(jnp.where(gt, lm, 0.0), axis=1, keepdims=True)
        topk = sum_gt + (kf - cnt_gt) * tau
        num_matched = jnp.sum(npb)
        conf_loss = sc_s[1, 0] + jnp.sum(topk)
        out_ref[0, 0] = (sc_s[0, 0] + conf_loss) / num_matched


def kernel(loc_preds, loc_targets, conf_preds, conf_targets):
    lpt = jnp.stack([loc_preds, loc_targets], axis=1).reshape(
        _B, 2, _LR, _LC)
    ct59 = conf_targets.reshape(_B, _LR, _CR)

    def ix3(i):
        return (jnp.minimum(i, _B - 1), 0, 0)

    def ix4(i):
        return (jnp.minimum(i, _B - 1), 0, 0, 0)

    loss = pl.pallas_call(
        _fused,
        grid=(_B + 1,),
        in_specs=[
            pl.BlockSpec((1, _N, _C), ix3),
            pl.BlockSpec((_B, _N), lambda i: (0, 0)),
            pl.BlockSpec((1, 2, _LR, _LC), ix4),
            pl.BlockSpec((1, _LR, _CR), ix3),
        ],
        out_specs=pl.BlockSpec(memory_space=pltpu.SMEM),
        out_shape=jax.ShapeDtypeStruct((1, 1), jnp.float32),
        scratch_shapes=[
            pltpu.VMEM((_B, _NP), jnp.float32),
            pltpu.VMEM((8, 128), jnp.float32),
        ],
    )(conf_preds, conf_targets, lpt, ct59)
    return loss[0, 0]


# speedup vs baseline: 1.0596x; 1.0270x over previous
import jax
import jax.numpy as jnp
from jax import lax
from jax.experimental import pallas as pl
from jax.experimental.pallas import tpu as pltpu

_B, _N, _C = 32, 8732, 21
_NP = 8832
_LR, _LC = 148, 236
_CR = _LC // 4
_INF_BITS = 0x7F800000


def _fused(conf_ref, ct_ref, lpt_ref, ct59_ref, out_ref, lm_s, sc_s):
    i = pl.program_id(0)

    @pl.when(i == 0)
    def _zero():
        lm_s[...] = jnp.zeros((_B, _NP), jnp.float32)
        sc_s[...] = jnp.zeros((8, 128), jnp.float32)

    @pl.when(i < _B // 2)
    def _stage1():
        hi = lax.Precision.DEFAULT
        sub = lax.broadcasted_iota(jnp.int32, (_C, _N), 0)
        ones_row = jnp.ones((1, _C), jnp.float32)
        r = (lax.broadcasted_iota(jnp.int32, (_LC, _CR), 0) // 4
             == lax.broadcasted_iota(jnp.int32, (_LC, _CR), 1)
             ).astype(jnp.float32)
        row = lax.broadcasted_iota(jnp.int32, (8, 128), 0)

        for rr in range(2):
            b = 2 * i + rr
            ct_row = ct_ref[pl.ds(b, 1), :]
            posf = (ct_row > 0).astype(jnp.float32)

            Xt = jnp.transpose(conf_ref[rr])
            Em = jnp.exp(Xt)
            Z = jnp.dot(ones_row, Em, precision=hi)
            pick = jnp.dot(ones_row,
                           jnp.where(sub == ct_row, Xt, 0.0),
                           precision=hi)
            ce = jnp.log(Z) - pick
            lm_s[pl.ds(b, 1), 0:_N] = jnp.where(posf > 0, 0.0, ce)
            spce_b = jnp.sum(ce * posf)

            ld = lpt_ref[rr, 0] - lpt_ref[rr, 1]
            ad = jnp.abs(ld)
            y = jnp.where(ad < 1.0, 0.5 * ld * ld, ad - 0.5)
            s59 = jnp.dot(y, r, precision=hi)
            posf59 = (ct59_ref[rr] > 0).astype(jnp.float32)
            loc_b = jnp.sum(s59 * posf59)

            sc_s[...] += (jnp.where(row == 0, loc_b, 0.0)
                          + jnp.where(row == 1, spce_b, 0.0))

    @pl.when(i == _B // 2)
    def _stage2():
        lm = lm_s[...]
        posm = (ct_ref[...] > 0).astype(jnp.float32)
        npb = jnp.sum(posm, axis=1, keepdims=True)
        kf = jnp.clip(3.0 * npb, 1.0, float(_N - 1))
        bits = lax.bitcast_convert_type(lm, jnp.int32)

        def step(_, lohi):
            lo, hi2 = lohi
            mid = lo + ((hi2 - lo + 1) >> 1)
            cnt = jnp.sum((bits >= mid).astype(jnp.float32), axis=1,
                          keepdims=True)
            ge = cnt >= kf
            return jnp.where(ge, mid, lo), jnp.where(ge, hi2, mid - 1)

        lo0 = jnp.zeros((_B, 1), jnp.int32)
        hi0 = jnp.full((_B, 1), _INF_BITS, jnp.int32)
        lo, _ = lax.fori_loop(0, 31, step, (lo0, hi0))
        tau = lax.bitcast_convert_type(lo, jnp.float32)

        gt = lm > tau
        cnt_gt = jnp.sum(gt.astype(jnp.float32), axis=1, keepdims=True)
        sum_gt = jnp.sum(jnp.where(gt, lm, 0.0), axis=1, keepdims=True)
        topk = sum_gt + (kf - cnt_gt) * tau
        num_matched = jnp.sum(npb)
        conf_loss = sc_s[1, 0] + jnp.sum(topk)
        out_ref[0, 0] = (sc_s[0, 0] + conf_loss) / num_matched


def kernel(loc_preds, loc_targets, conf_preds, conf_targets):
    lpt = jnp.stack([loc_preds, loc_targets], axis=1).reshape(
        _B, 2, _LR, _LC)
    ct59 = conf_targets.reshape(_B, _LR, _CR)

    def ix3(i):
        return (jnp.minimum(i, _B // 2 - 1), 0, 0)

    def ix4(i):
        return (jnp.minimum(i, _B // 2 - 1), 0, 0, 0)

    loss = pl.pallas_call(
        _fused,
        grid=(_B // 2 + 1,),
        in_specs=[
            pl.BlockSpec((2, _N, _C), ix3),
            pl.BlockSpec((_B, _N), lambda i: (0, 0)),
            pl.BlockSpec((2, 2, _LR, _LC), ix4),
            pl.BlockSpec((2, _LR, _CR), ix3),
        ],
        out_specs=pl.BlockSpec(memory_space=pltpu.SMEM),
        out_shape=jax.ShapeDtypeStruct((1, 1), jnp.float32),
        scratch_shapes=[
            pltpu.VMEM((_B, _NP), jnp.float32),
            pltpu.VMEM((8, 128), jnp.float32),
        ],
    )(conf_preds, conf_targets, lpt, ct59)
    return loss[0, 0]
